# baseline (speedup 1.0000x reference)
import functools

import jax
import jax.numpy as jnp
from jax import lax
from jax.experimental import pallas as pl
from jax.experimental.pallas import tpu as pltpu
from jax.experimental.pallas import tpu_sc as plsc

N_LAYERS = 24
K = 6
OUT_NUM = 2
INV_TEMP = 2.0
B = 8
PADDED = 32


def _tree(op, xs):
    while len(xs) > 1:
        xs = [op(xs[i], xs[i + 1]) for i in range(0, len(xs) - 1, 2)] + (
            [xs[-1]] if len(xs) % 2 else [])
    return xs[0]


def _gate_body(scores_hbm, out_hbm, col_v, out_v):
    c = lax.axis_index("s")
    pltpu.sync_copy(scores_hbm.at[c], col_v)
    v0 = col_v[pl.ds(0, 16)]
    v1 = col_v[pl.ds(16, 16)]
    iota = lax.iota(jnp.int32, 16)
    one = jnp.full((16,), 1.0, jnp.float32)
    zero = jnp.full((16,), 0.0, jnp.float32)
    sj = [v0[j] if j < 16 else v1[j - 16] for j in range(N_LAYERS)]
    rank0 = zero
    rank1 = zero
    for j in range(N_LAYERS):
        bj = one * sj[j]
        rank0 = rank0 + jnp.where(
            (bj > v0) | ((bj == v0) & (iota > j)), one, zero)
        rank1 = rank1 + jnp.where(
            (bj > v1) | ((bj == v1) & (iota + 16 > j)), one, zero)
    valid1 = iota < (N_LAYERS - 16)
    keep0 = rank0 < float(K)
    keep1 = (rank1 < float(K)) & valid1
    m = _tree(jnp.maximum, sj)
    e0 = jnp.where(keep0, jnp.exp((v0 - m) * INV_TEMP), zero)
    e1 = jnp.where(keep1, jnp.exp((jnp.where(valid1, v1, m) - m) * INV_TEMP),
                   zero)
    lanes = [e0[j] for j in range(16)] + [e1[j] for j in range(8)]
    total = one * _tree(jnp.add, lanes)
    g0 = e0 / total
    g1 = e1 / total
    for b in range(B):
        out_v[pl.ds(b * PADDED, 16)] = g0
        out_v[pl.ds(b * PADDED + 16, 16)] = g1
    pltpu.sync_copy(out_v, out_hbm.at[c])


_gate_kernel = functools.partial(
    pl.kernel,
    out_type=jax.ShapeDtypeStruct((OUT_NUM, B * PADDED), jnp.float32),
    mesh=plsc.VectorSubcoreMesh(
        core_axis_name="c", subcore_axis_name="s",
        num_cores=1, num_subcores=OUT_NUM),
    scratch_types=[
        pltpu.VMEM((PADDED,), jnp.float32),
        pltpu.VMEM((B * PADDED,), jnp.float32),
    ],
)(_gate_body)


def kernel(X, scores):
    del X
    st = jnp.pad(scores.T, ((0, 0), (0, PADDED - N_LAYERS)))
    out = _gate_kernel(st).reshape(OUT_NUM, B, PADDED)
    return out[:, :, :N_LAYERS].transpose(1, 2, 0)
